# baseline (speedup 1.0000x reference)
import functools
import math

import jax
import jax.numpy as jnp
from jax import lax
from jax.experimental import pallas as pl
from jax.experimental.pallas import tpu as pltpu
from jax.experimental.pallas import tpu_sc as plsc

_DS, _DA, _DR = 64, 32, 32
_DM = _DS + _DA + _DR
_NF = 26
_B, _L = 4096, 50
_N = _B * _L
_EPS = 1e-6

_NC, _NSUB = 2, 16
_NW = _NC * _NSUB
_IDS_PER_GATHER = 128
_CROWS = 5
_CTOK = _CROWS * _IDS_PER_GATHER
_TPW = _N // _NW
_NCH = _TPW // _CTOK


def _sc_gather(sid, aid, rid, song_table, album_table, artist_table):
    mesh = plsc.VectorSubcoreMesh(core_axis_name="c", subcore_axis_name="s")

    @functools.partial(
        pl.kernel,
        mesh=mesh,
        compiler_params=pltpu.CompilerParams(use_tc_tiling_on_sc=False),
        out_type=jax.ShapeDtypeStruct((_N, _DM), jnp.float32),
        scratch_types=[
            pltpu.VMEM((_CTOK,), jnp.int32),
            pltpu.VMEM((_CTOK,), jnp.int32),
            pltpu.VMEM((_CTOK,), jnp.int32),
            pltpu.VMEM((_CTOK, _DS), jnp.float32),
            pltpu.VMEM((_CTOK, _DA), jnp.float32),
            pltpu.VMEM((_CTOK, _DR), jnp.float32),
            pltpu.SemaphoreType.DMA,
        ],
    )
    def k(sid_h, aid_h, rid_h, st_h, at_h, rt_h, cat_h,
          idx_s, idx_a, idx_r, rows_s, rows_a, rows_r, sem):
        wid = lax.axis_index("s") * _NC + lax.axis_index("c")

        def body(c, carry):
            tbase = (wid * _NCH + c) * _CTOK
            pltpu.sync_copy(sid_h.at[pl.ds(tbase, _CTOK)], idx_s)
            pltpu.sync_copy(aid_h.at[pl.ds(tbase, _CTOK)], idx_a)
            pltpu.sync_copy(rid_h.at[pl.ds(tbase, _CTOK)], idx_r)
            handles = []
            for j in range(_CROWS):
                sl = pl.ds(j * _IDS_PER_GATHER, _IDS_PER_GATHER)
                handles.append(pltpu.async_copy(st_h.at[idx_s.at[sl]], rows_s.at[sl], sem))
                handles.append(pltpu.async_copy(at_h.at[idx_a.at[sl]], rows_a.at[sl], sem))
                handles.append(pltpu.async_copy(rt_h.at[idx_r.at[sl]], rows_r.at[sl], sem))
            for h in handles:
                h.wait()
            rows = pl.ds(tbase, _CTOK)
            pltpu.sync_copy(rows_s, cat_h.at[rows, pl.ds(0, _DS)])
            pltpu.sync_copy(rows_a, cat_h.at[rows, pl.ds(_DS, _DA)])
            pltpu.sync_copy(rows_r, cat_h.at[rows, pl.ds(_DS + _DA, _DR)])
            return carry

        lax.fori_loop(0, _NCH, body, 0)

    return k(sid, aid, rid, song_table, album_table, artist_table)


def _tc_fused(cat, f2, W, b, alpha, bias):
    TB = 2048
    scale = math.sqrt(_DM)

    def body(cat_ref, f_ref, w_ref, b_ref, al_ref, bi_ref, o_ref):
        en = lax.dot_general(f_ref[...], w_ref[...],
                             (((0,), (0,)), ((), ())),
                             preferred_element_type=jnp.float32)
        y = cat_ref[...] * scale + en + b_ref[...]
        mean = jnp.mean(y, axis=-1, keepdims=True)
        d = y - mean
        var = jnp.sum(d * d, axis=-1, keepdims=True) * (1.0 / (_DM - 1))
        o_ref[...] = al_ref[...] * d / (jnp.sqrt(var) + _EPS) + bi_ref[...]

    return pl.pallas_call(
        body,
        grid=(_N // TB,),
        compiler_params=pltpu.CompilerParams(vmem_limit_bytes=50 * 1024 * 1024),
        in_specs=[
            pl.BlockSpec((TB, _DM), lambda i: (i, 0)),
            pl.BlockSpec((_NF, TB), lambda i: (0, i)),
            pl.BlockSpec((_NF, _DM), lambda i: (0, 0)),
            pl.BlockSpec((1, _DM), lambda i: (0, 0)),
            pl.BlockSpec((1, _DM), lambda i: (0, 0)),
            pl.BlockSpec((1, _DM), lambda i: (0, 0)),
        ],
        out_specs=pl.BlockSpec((TB, _DM), lambda i: (i, 0)),
        out_shape=jax.ShapeDtypeStruct((_N, _DM), jnp.float32),
    )(cat, f2, W, b, alpha, bias)


def kernel(song_ids, album_ids, artist_ids, num_features,
           song_table, album_table, artist_table, W_num, b_num, alpha, bias):
    sid = song_ids.T.reshape(_N)
    aid = album_ids.T.reshape(_N)
    rid = artist_ids.T.reshape(_N)
    featsT = num_features.transpose(2, 1, 0)
    st = lax.optimization_barrier(song_table.reshape(500000, 128))
    at_ = lax.optimization_barrier(album_table.reshape(25000, 128))
    rt = lax.optimization_barrier(artist_table.reshape(25000, 128))
    cat = _sc_gather(sid, aid, rid,
                     st.reshape(1000000, 64),
                     at_.reshape(100000, 32),
                     rt.reshape(100000, 32))
    out_flat = _tc_fused(cat, featsT.reshape(_NF, _N), W_num,
                         b_num.reshape(1, _DM), alpha.reshape(1, _DM),
                         bias.reshape(1, _DM))
    return out_flat.reshape(_L, _B, _DM).transpose(1, 0, 2)


# speedup vs baseline: 1.0112x; 1.0112x over previous
import functools
import math

import jax
import jax.numpy as jnp
from jax import lax
from jax.experimental import pallas as pl
from jax.experimental.pallas import tpu as pltpu
from jax.experimental.pallas import tpu_sc as plsc

_DS, _DA, _DR = 64, 32, 32
_DM = _DS + _DA + _DR
_NF = 26
_B, _L = 4096, 50
_N = _B * _L
_EPS = 1e-6

_NC, _NSUB = 2, 16
_NW = _NC * _NSUB
_IDS_PER_GATHER = 128
_CROWS = 5
_CTOK = _CROWS * _IDS_PER_GATHER
_TPW = _N // _NW
_NCH = _TPW // _CTOK


def _sc_gather(sid, aid, rid, song_table, album_table, artist_table):
    mesh = plsc.VectorSubcoreMesh(core_axis_name="c", subcore_axis_name="s")

    @functools.partial(
        pl.kernel,
        mesh=mesh,
        compiler_params=pltpu.CompilerParams(use_tc_tiling_on_sc=False),
        out_type=jax.ShapeDtypeStruct((_N, _DM), jnp.float32),
        scratch_types=[
            pltpu.VMEM((_CTOK,), jnp.int32),
            pltpu.VMEM((_CTOK,), jnp.int32),
            pltpu.VMEM((_CTOK,), jnp.int32),
            pltpu.VMEM((_CTOK, 2 * _DS), jnp.float32),
            pltpu.VMEM((_CTOK, _DA), jnp.float32),
            pltpu.VMEM((_CTOK, _DR), jnp.float32),
            pltpu.SemaphoreType.DMA,
        ],
    )
    def k(sid_h, aid_h, rid_h, st_h, at_h, rt_h, cat_h,
          idx_s, idx_a, idx_r, rows_s, rows_a, rows_r, sem):
        wid = lax.axis_index("s") * _NC + lax.axis_index("c")

        def body(c, carry):
            tbase = (wid * _NCH + c) * _CTOK
            pltpu.sync_copy(sid_h.at[pl.ds(tbase, _CTOK)], idx_s)
            pltpu.sync_copy(aid_h.at[pl.ds(tbase, _CTOK)], idx_a)
            pltpu.sync_copy(rid_h.at[pl.ds(tbase, _CTOK)], idx_r)
            handles = []
            for j in range(_CROWS):
                sl = pl.ds(j * _IDS_PER_GATHER, _IDS_PER_GATHER)
                handles.append(pltpu.async_copy(st_h.at[idx_s.at[sl]], rows_s.at[sl], sem))
                handles.append(pltpu.async_copy(at_h.at[idx_a.at[sl]], rows_a.at[sl], sem))
                handles.append(pltpu.async_copy(rt_h.at[idx_r.at[sl]], rows_r.at[sl], sem))
            for h in handles:
                h.wait()
            rows = pl.ds(tbase, _CTOK)
            pltpu.sync_copy(rows_s.at[:, pl.ds(0, _DS)], cat_h.at[rows, pl.ds(0, _DS)])
            pltpu.sync_copy(rows_a, cat_h.at[rows, pl.ds(_DS, _DA)])
            pltpu.sync_copy(rows_r, cat_h.at[rows, pl.ds(_DS + _DA, _DR)])
            return carry

        lax.fori_loop(0, _NCH, body, 0)

    return k(sid, aid, rid, song_table, album_table, artist_table)


def _tc_pack(tableT, R, D):
    CB = 2048

    def body(in_ref, o_ref):
        xt = jnp.swapaxes(in_ref[...], 0, 1)
        o_ref[...] = jnp.concatenate([xt, xt], axis=1)

    return pl.pallas_call(
        body,
        grid=(pl.cdiv(R, CB),),
        in_specs=[pl.BlockSpec((D, CB), lambda i: (0, i))],
        out_specs=pl.BlockSpec((CB, 2 * D), lambda i: (i, 0)),
        out_shape=jax.ShapeDtypeStruct((R, 2 * D), jnp.float32),
    )(tableT)


def _tc_fused(cat, f2, W, b, alpha, bias):
    TB = 2048
    scale = math.sqrt(_DM)

    def body(cat_ref, f_ref, w_ref, b_ref, al_ref, bi_ref, o_ref):
        en = lax.dot_general(f_ref[...], w_ref[...],
                             (((0,), (0,)), ((), ())),
                             preferred_element_type=jnp.float32)
        y = cat_ref[...] * scale + en + b_ref[...]
        mean = jnp.mean(y, axis=-1, keepdims=True)
        d = y - mean
        var = jnp.sum(d * d, axis=-1, keepdims=True) * (1.0 / (_DM - 1))
        o_ref[...] = al_ref[...] * d / (jnp.sqrt(var) + _EPS) + bi_ref[...]

    return pl.pallas_call(
        body,
        grid=(_N // TB,),
        compiler_params=pltpu.CompilerParams(vmem_limit_bytes=50 * 1024 * 1024),
        in_specs=[
            pl.BlockSpec((TB, _DM), lambda i: (i, 0)),
            pl.BlockSpec((_NF, TB), lambda i: (0, i)),
            pl.BlockSpec((_NF, _DM), lambda i: (0, 0)),
            pl.BlockSpec((1, _DM), lambda i: (0, 0)),
            pl.BlockSpec((1, _DM), lambda i: (0, 0)),
            pl.BlockSpec((1, _DM), lambda i: (0, 0)),
        ],
        out_specs=pl.BlockSpec((TB, _DM), lambda i: (i, 0)),
        out_shape=jax.ShapeDtypeStruct((_N, _DM), jnp.float32),
    )(cat, f2, W, b, alpha, bias)


def kernel(song_ids, album_ids, artist_ids, num_features,
           song_table, album_table, artist_table, W_num, b_num, alpha, bias):
    sid = song_ids.T.reshape(_N)
    aid = album_ids.T.reshape(_N)
    rid = artist_ids.T.reshape(_N)
    featsT = num_features.transpose(2, 1, 0)
    st = _tc_pack(song_table.T, 1000000, _DS)
    at_ = lax.optimization_barrier(album_table.reshape(25000, 128))
    rt = lax.optimization_barrier(artist_table.reshape(25000, 128))
    cat = _sc_gather(sid, aid, rid, st,
                     at_.reshape(100000, 32),
                     rt.reshape(100000, 32))
    out_flat = _tc_fused(cat, featsT.reshape(_NF, _N), W_num,
                         b_num.reshape(1, _DM), alpha.reshape(1, _DM),
                         bias.reshape(1, _DM))
    return out_flat.reshape(_L, _B, _DM).transpose(1, 0, 2)


# speedup vs baseline: 1.2744x; 1.2602x over previous
import functools
import math

import jax
import jax.numpy as jnp
from jax import lax
from jax.experimental import pallas as pl
from jax.experimental.pallas import tpu as pltpu
from jax.experimental.pallas import tpu_sc as plsc

_DS, _DA, _DR = 64, 32, 32
_DM = _DS + _DA + _DR
_NF = 26
_B, _L = 4096, 50
_N = _B * _L
_EPS = 1e-6

_NC, _NSUB = 2, 16
_NW = _NC * _NSUB
_IDS_PER_GATHER = 128
_CROWS = 5
_CTOK = _CROWS * _IDS_PER_GATHER
_TPW = _N // _NW
_NCH = _TPW // _CTOK


def _sc_gather(sid, aid, rid, song_table, album_table, artist_table):
    mesh = plsc.VectorSubcoreMesh(core_axis_name="c", subcore_axis_name="s")

    @functools.partial(
        pl.kernel,
        mesh=mesh,
        compiler_params=pltpu.CompilerParams(use_tc_tiling_on_sc=False),
        out_type=jax.ShapeDtypeStruct((_N, _DM), jnp.float32),
        scratch_types=[
            pltpu.VMEM((_CTOK,), jnp.int32),
            pltpu.VMEM((_CTOK,), jnp.int32),
            pltpu.VMEM((_CTOK,), jnp.int32),
            pltpu.VMEM((_CTOK, 2 * _DS), jnp.float32),
            pltpu.VMEM((_CTOK, _DA), jnp.float32),
            pltpu.VMEM((_CTOK, _DR), jnp.float32),
            pltpu.SemaphoreType.DMA,
        ],
    )
    def k(sid_h, aid_h, rid_h, st_h, at_h, rt_h, cat_h,
          idx_s, idx_a, idx_r, rows_s, rows_a, rows_r, sem):
        wid = lax.axis_index("s") * _NC + lax.axis_index("c")

        def body(c, carry):
            tbase = (wid * _NCH + c) * _CTOK
            pltpu.sync_copy(sid_h.at[pl.ds(tbase, _CTOK)], idx_s)
            pltpu.sync_copy(aid_h.at[pl.ds(tbase, _CTOK)], idx_a)
            pltpu.sync_copy(rid_h.at[pl.ds(tbase, _CTOK)], idx_r)
            handles = []
            for j in range(_CROWS):
                sl = pl.ds(j * _IDS_PER_GATHER, _IDS_PER_GATHER)
                handles.append(pltpu.async_copy(st_h.at[idx_s.at[sl]], rows_s.at[sl], sem))
                handles.append(pltpu.async_copy(at_h.at[idx_a.at[sl]], rows_a.at[sl], sem))
                handles.append(pltpu.async_copy(rt_h.at[idx_r.at[sl]], rows_r.at[sl], sem))
            for h in handles:
                h.wait()
            rows = pl.ds(tbase, _CTOK)
            pltpu.sync_copy(rows_s.at[:, pl.ds(0, _DS)], cat_h.at[rows, pl.ds(0, _DS)])
            pltpu.sync_copy(rows_a, cat_h.at[rows, pl.ds(_DS, _DA)])
            pltpu.sync_copy(rows_r, cat_h.at[rows, pl.ds(_DS + _DA, _DR)])
            return carry

        lax.fori_loop(0, _NCH, body, 0)

    return k(sid, aid, rid, song_table, album_table, artist_table)


def _tc_pack(tableT, R, D):
    CB = 8192

    def body(in_ref, o_ref):
        xt = jnp.swapaxes(in_ref[...], 0, 1)
        o_ref[...] = jnp.concatenate([xt, xt], axis=1)

    return pl.pallas_call(
        body,
        grid=(pl.cdiv(R, CB),),
        in_specs=[pl.BlockSpec((D, CB), lambda i: (0, i))],
        out_specs=pl.BlockSpec((CB, 2 * D), lambda i: (i, 0)),
        out_shape=jax.ShapeDtypeStruct((R, 2 * D), jnp.float32),
    )(tableT)


def _tc_fused(cat, f2, W, b, alpha, bias):
    TB = 2048
    scale = math.sqrt(_DM)

    def body(cat_ref, f_ref, w_ref, b_ref, al_ref, bi_ref, o_ref):
        en = lax.dot_general(f_ref[...], w_ref[...],
                             (((0,), (0,)), ((), ())),
                             preferred_element_type=jnp.float32)
        y = cat_ref[...] * scale + en + b_ref[...]
        mean = jnp.mean(y, axis=-1, keepdims=True)
        d = y - mean
        var = jnp.sum(d * d, axis=-1, keepdims=True) * (1.0 / (_DM - 1))
        o_ref[...] = al_ref[...] * d / (jnp.sqrt(var) + _EPS) + bi_ref[...]

    return pl.pallas_call(
        body,
        grid=(_N // TB,),
        compiler_params=pltpu.CompilerParams(vmem_limit_bytes=50 * 1024 * 1024),
        in_specs=[
            pl.BlockSpec((TB, _DM), lambda i: (i, 0)),
            pl.BlockSpec((_NF, TB), lambda i: (0, i)),
            pl.BlockSpec((_NF, _DM), lambda i: (0, 0)),
            pl.BlockSpec((1, _DM), lambda i: (0, 0)),
            pl.BlockSpec((1, _DM), lambda i: (0, 0)),
            pl.BlockSpec((1, _DM), lambda i: (0, 0)),
        ],
        out_specs=pl.BlockSpec((TB, _DM), lambda i: (i, 0)),
        out_shape=jax.ShapeDtypeStruct((_N, _DM), jnp.float32),
    )(cat, f2, W, b, alpha, bias)


def kernel(song_ids, album_ids, artist_ids, num_features,
           song_table, album_table, artist_table, W_num, b_num, alpha, bias):
    sid = song_ids.T.reshape(_N)
    aid = album_ids.T.reshape(_N)
    rid = artist_ids.T.reshape(_N)
    featsT = num_features.transpose(2, 1, 0)
    st = _tc_pack(song_table.T, 1000000, _DS)
    at_ = lax.optimization_barrier(album_table.reshape(25000, 128))
    rt = lax.optimization_barrier(artist_table.reshape(25000, 128))
    cat = _sc_gather(sid, aid, rid, st,
                     at_.reshape(100000, 32),
                     rt.reshape(100000, 32))
    out_flat = _tc_fused(cat, featsT.reshape(_NF, _N), W_num,
                         b_num.reshape(1, _DM), alpha.reshape(1, _DM),
                         bias.reshape(1, _DM))
    return out_flat.reshape(_L, _B, _DM).transpose(1, 0, 2)


# speedup vs baseline: 1.3382x; 1.0501x over previous
import functools
import math

import jax
import jax.numpy as jnp
from jax import lax
from jax.experimental import pallas as pl
from jax.experimental.pallas import tpu as pltpu
from jax.experimental.pallas import tpu_sc as plsc

_DS, _DA, _DR = 64, 32, 32
_DM = _DS + _DA + _DR
_NF = 26
_B, _L = 4096, 50
_N = _B * _L
_EPS = 1e-6

_NC, _NSUB = 2, 16
_NW = _NC * _NSUB
_IDS_PER_GATHER = 128
_CROWS = 5
_CTOK = _CROWS * _IDS_PER_GATHER
_TPW = _N // _NW
_NCH = _TPW // _CTOK


def _sc_gather(sid, aid, rid, song_table, album_table, artist_table):
    mesh = plsc.VectorSubcoreMesh(core_axis_name="c", subcore_axis_name="s")

    @functools.partial(
        pl.kernel,
        mesh=mesh,
        compiler_params=pltpu.CompilerParams(use_tc_tiling_on_sc=False),
        out_type=jax.ShapeDtypeStruct((_N, _DM), jnp.float32),
        scratch_types=[
            pltpu.VMEM((_CTOK,), jnp.int32),
            pltpu.VMEM((_CTOK,), jnp.int32),
            pltpu.VMEM((_CTOK,), jnp.int32),
            pltpu.VMEM((_CTOK, 2 * _DS), jnp.float32),
            pltpu.VMEM((_CTOK, _DA), jnp.float32),
            pltpu.VMEM((_CTOK, _DR), jnp.float32),
            pltpu.SemaphoreType.DMA,
        ],
    )
    def k(sid_h, aid_h, rid_h, st_h, at_h, rt_h, cat_h,
          idx_s, idx_a, idx_r, rows_s, rows_a, rows_r, sem):
        wid = lax.axis_index("s") * _NC + lax.axis_index("c")

        def body(c, carry):
            tbase = (wid * _NCH + c) * _CTOK
            pltpu.sync_copy(sid_h.at[pl.ds(tbase, _CTOK)], idx_s)
            pltpu.sync_copy(aid_h.at[pl.ds(tbase, _CTOK)], idx_a)
            pltpu.sync_copy(rid_h.at[pl.ds(tbase, _CTOK)], idx_r)
            handles = []
            for j in range(_CROWS):
                sl = pl.ds(j * _IDS_PER_GATHER, _IDS_PER_GATHER)
                handles.append(pltpu.async_copy(st_h.at[idx_s.at[sl]], rows_s.at[sl], sem))
                handles.append(pltpu.async_copy(at_h.at[idx_a.at[sl]], rows_a.at[sl], sem))
                handles.append(pltpu.async_copy(rt_h.at[idx_r.at[sl]], rows_r.at[sl], sem))
            for h in handles:
                h.wait()
            rows = pl.ds(tbase, _CTOK)
            pltpu.sync_copy(rows_s.at[:, pl.ds(0, _DS)], cat_h.at[rows, pl.ds(0, _DS)])
            pltpu.sync_copy(rows_a, cat_h.at[rows, pl.ds(_DS, _DA)])
            pltpu.sync_copy(rows_r, cat_h.at[rows, pl.ds(_DS + _DA, _DR)])
            return carry

        lax.fori_loop(0, _NCH, body, 0)

    return k(sid, aid, rid, song_table, album_table, artist_table)


def _tc_pack(tableT, R, D):
    CB = 16384

    def body(in_ref, o_ref):
        xt = jnp.swapaxes(in_ref[...], 0, 1)
        o_ref[...] = jnp.concatenate([xt, xt], axis=1)

    return pl.pallas_call(
        body,
        grid=(pl.cdiv(R, CB),),
        in_specs=[pl.BlockSpec((D, CB), lambda i: (0, i))],
        out_specs=pl.BlockSpec((CB, 2 * D), lambda i: (i, 0)),
        out_shape=jax.ShapeDtypeStruct((R, 2 * D), jnp.float32),
    )(tableT)


def _tc_fused(cat, f2, W, b, alpha, bias):
    TB = 2048
    scale = math.sqrt(_DM)

    def body(cat_ref, f_ref, w_ref, b_ref, al_ref, bi_ref, o_ref):
        en = lax.dot_general(f_ref[...], w_ref[...],
                             (((0,), (0,)), ((), ())),
                             preferred_element_type=jnp.float32)
        y = cat_ref[...] * scale + en + b_ref[...]
        mean = jnp.mean(y, axis=-1, keepdims=True)
        d = y - mean
        var = jnp.sum(d * d, axis=-1, keepdims=True) * (1.0 / (_DM - 1))
        o_ref[...] = al_ref[...] * d / (jnp.sqrt(var) + _EPS) + bi_ref[...]

    return pl.pallas_call(
        body,
        grid=(_N // TB,),
        compiler_params=pltpu.CompilerParams(vmem_limit_bytes=50 * 1024 * 1024),
        in_specs=[
            pl.BlockSpec((TB, _DM), lambda i: (i, 0)),
            pl.BlockSpec((_NF, TB), lambda i: (0, i)),
            pl.BlockSpec((_NF, _DM), lambda i: (0, 0)),
            pl.BlockSpec((1, _DM), lambda i: (0, 0)),
            pl.BlockSpec((1, _DM), lambda i: (0, 0)),
            pl.BlockSpec((1, _DM), lambda i: (0, 0)),
        ],
        out_specs=pl.BlockSpec((TB, _DM), lambda i: (i, 0)),
        out_shape=jax.ShapeDtypeStruct((_N, _DM), jnp.float32),
    )(cat, f2, W, b, alpha, bias)


def kernel(song_ids, album_ids, artist_ids, num_features,
           song_table, album_table, artist_table, W_num, b_num, alpha, bias):
    sid = song_ids.T.reshape(_N)
    aid = album_ids.T.reshape(_N)
    rid = artist_ids.T.reshape(_N)
    featsT = num_features.transpose(2, 1, 0)
    st = _tc_pack(song_table.T, 1000000, _DS)
    at_ = lax.optimization_barrier(album_table.reshape(25000, 128))
    rt = lax.optimization_barrier(artist_table.reshape(25000, 128))
    cat = _sc_gather(sid, aid, rid, st,
                     at_.reshape(100000, 32),
                     rt.reshape(100000, 32))
    out_flat = _tc_fused(cat, featsT.reshape(_NF, _N), W_num,
                         b_num.reshape(1, _DM), alpha.reshape(1, _DM),
                         bias.reshape(1, _DM))
    return out_flat.reshape(_L, _B, _DM).transpose(1, 0, 2)
